# baseline (speedup 1.0000x reference)
import functools
import math

import jax
import jax.numpy as jnp
from jax import lax
from jax.experimental import pallas as pl
from jax.experimental.pallas import tpu as pltpu
from jax.experimental.pallas import tpu_sc as plsc

DIM_EMB = 64
SCALE = math.sqrt(DIM_EMB)

_info = plsc.get_sparse_core_info()
_NC, _NS, _L = _info.num_cores, _info.num_subcores, _info.num_lanes
_NW = _NC * _NS


def _emb_kernel(b_per_w: int, chunk: int, table_hbm, idx_hbm, out_hbm,
                idx_v, rows_v, sem):
    wid = lax.axis_index("s") * _NC + lax.axis_index("c")
    base = wid * b_per_w
    n_chunks = b_per_w // chunk
    n_vec = DIM_EMB // _L

    def chunk_body(g, _):
        off = base + g * chunk
        pltpu.sync_copy(idx_hbm.at[pl.ds(off, chunk)], idx_v)
        pltpu.async_copy(table_hbm.at[idx_v], rows_v, sem).wait()

        def scale_body(r, _):
            for c in range(n_vec):
                sl = (r, pl.ds(c * _L, _L))
                rows_v[sl] = rows_v[sl] * SCALE
            return 0

        lax.fori_loop(0, chunk, scale_body, 0, unroll=8)
        pltpu.sync_copy(rows_v, out_hbm.at[pl.ds(off, chunk)])
        return 0

    lax.fori_loop(0, n_chunks, chunk_body, 0)


def kernel(tokens, table):
    n_seq, n_tok = tokens.shape
    B = n_seq * n_tok
    b_per_w = B // _NW
    chunk = 1024
    idx = tokens.reshape(B).astype(jnp.int32)

    mesh = plsc.VectorSubcoreMesh(core_axis_name="c", subcore_axis_name="s")
    k = pl.kernel(
        functools.partial(_emb_kernel, b_per_w, chunk),
        mesh=mesh,
        out_type=jax.ShapeDtypeStruct((B, DIM_EMB), jnp.float32),
        scratch_types=[
            pltpu.VMEM((chunk,), jnp.int32),
            pltpu.VMEM((chunk, DIM_EMB), jnp.float32),
            pltpu.SemaphoreType.DMA,
        ],
        compiler_params=pltpu.CompilerParams(use_tc_tiling_on_sc=False),
    )
    out = k(table, idx)
    return out.reshape(n_seq, n_tok, DIM_EMB)


# speedup vs baseline: 1.0348x; 1.0348x over previous
import functools
import math

import jax
import jax.numpy as jnp
from jax import lax
from jax.experimental import pallas as pl
from jax.experimental.pallas import tpu as pltpu
from jax.experimental.pallas import tpu_sc as plsc

DIM_EMB = 64
SCALE = math.sqrt(DIM_EMB)

_info = plsc.get_sparse_core_info()
_NC, _NS, _L = _info.num_cores, _info.num_subcores, _info.num_lanes
_NW = _NC * _NS
_NBUF = 4
_DEPTH = 2


def _emb_kernel(chunk: int, n_chunks: int, table_hbm, idx_hbm, out_hbm,
                idx_v, rows, gsems, osems):
    wid = lax.axis_index("s") * _NC + lax.axis_index("c")
    b_per_w = chunk * n_chunks
    base = wid * b_per_w
    n_vec = DIM_EMB // _L

    pltpu.sync_copy(idx_hbm.at[wid], idx_v)

    def start_gather(g, b):
        pltpu.async_copy(table_hbm.at[idx_v.at[g]], rows[b], gsems[b])

    for b in range(_DEPTH):
        start_gather(b, b)

    def quad_body(i, _):
        for b in range(_NBUF):
            g = i * _NBUF + b
            pltpu.make_async_copy(table_hbm.at[idx_v.at[g]], rows[b],
                                  gsems[b]).wait()

            def scale_body(r, _):
                for c in range(n_vec):
                    sl = (r, pl.ds(c * _L, _L))
                    rows[b][sl] = rows[b][sl] * SCALE
                return 0

            lax.fori_loop(0, chunk, scale_body, 0, unroll=8)

            dst = out_hbm.at[pl.ds(base + g * chunk, chunk)]
            pltpu.async_copy(rows[b], dst, osems[b])

            b2 = (b + _DEPTH) % _NBUF
            g2 = g + _DEPTH

            @pl.when(g2 < n_chunks)
            def _():
                @pl.when(g2 >= _NBUF)
                def _():
                    prev = g2 - _NBUF
                    pdst = out_hbm.at[pl.ds(base + prev * chunk, chunk)]
                    pltpu.make_async_copy(rows[b2], pdst, osems[b2]).wait()

                start_gather(g2, b2)

        return 0

    lax.fori_loop(0, n_chunks // _NBUF, quad_body, 0)

    for b in range(_NBUF):
        g = n_chunks - _NBUF + b
        dst = out_hbm.at[pl.ds(base + g * chunk, chunk)]
        pltpu.make_async_copy(rows[b], dst, osems[b]).wait()


def kernel(tokens, table):
    n_seq, n_tok = tokens.shape
    B = n_seq * n_tok
    b_per_w = B // _NW
    chunk = 320
    n_chunks = b_per_w // chunk
    idx = tokens.reshape(_NW, n_chunks, chunk).astype(jnp.int32)

    mesh = plsc.VectorSubcoreMesh(core_axis_name="c", subcore_axis_name="s")
    k = pl.kernel(
        functools.partial(_emb_kernel, chunk, n_chunks),
        mesh=mesh,
        out_type=jax.ShapeDtypeStruct((B, DIM_EMB), jnp.float32),
        scratch_types=[
            pltpu.VMEM((n_chunks, chunk), jnp.int32),
            [pltpu.VMEM((chunk, DIM_EMB), jnp.float32) for _ in range(_NBUF)],
            [pltpu.SemaphoreType.DMA for _ in range(_NBUF)],
            [pltpu.SemaphoreType.DMA for _ in range(_NBUF)],
        ],
        compiler_params=pltpu.CompilerParams(use_tc_tiling_on_sc=False),
    )
    out = k(table, idx)
    return out.reshape(n_seq, n_tok, DIM_EMB)
